# Initial kernel scaffold; baseline (speedup 1.0000x reference)
import jax
import jax.numpy as jnp
from jax.experimental import pallas as pl


def kernel(inp, connections, ram):
    raise NotImplementedError("write your pallas kernel here")


# speedup vs baseline: 31.7502x; 31.7502x over previous
import dataclasses
import functools

import jax
import jax.numpy as jnp
from jax import lax
from jax.experimental import pallas as pl
from jax.experimental.pallas import tpu as pltpu
from jax.experimental.pallas import tpu_sc as plsc

N_NEURONS = 6
N_BITS = 12
IN_BITS = 48
RAM_SIZE = 4096
TABLE = N_NEURONS * RAM_SIZE

BB = 8192
NW = 32


def _addr_body(conn_ref, inp_ref, out_ref):
    conn = conn_ref[...]
    ii = lax.broadcasted_iota(jnp.int32, (IN_BITS, N_NEURONS, N_BITS), 0)
    kk = lax.broadcasted_iota(jnp.int32, (IN_BITS, N_NEURONS, N_BITS), 2)
    pow2 = jnp.left_shift(jnp.int32(1), (N_BITS - 1) - kk)
    w = jnp.sum(jnp.where(conn[None, :, :] == ii, pow2, 0), axis=2)
    w_hi = (w >> 6).astype(jnp.bfloat16)
    w_lo = (w & 63).astype(jnp.bfloat16)

    x = inp_ref[...].astype(jnp.bfloat16)
    m_hi = jnp.dot(x, w_hi, preferred_element_type=jnp.float32)
    m_lo = jnp.dot(x, w_lo, preferred_element_type=jnp.float32)
    addr = (m_hi * 64.0 + m_lo).astype(jnp.int32)
    j = lax.broadcasted_iota(jnp.int32, (BB, N_NEURONS), 1)
    out_ref[...] = addr + j * RAM_SIZE


def _addresses(inp, connections):
    b = inp.shape[0]
    return pl.pallas_call(
        _addr_body,
        grid=(b // BB,),
        in_specs=[
            pl.BlockSpec((N_NEURONS, N_BITS), lambda i: (0, 0)),
            pl.BlockSpec((BB, IN_BITS), lambda i: (i, 0)),
        ],
        out_specs=pl.BlockSpec((BB, N_NEURONS), lambda i: (i, 0)),
        out_shape=jax.ShapeDtypeStruct((b, N_NEURONS), jnp.int32),
    )(connections, inp)


def _lookup(flat_idx, ram_flat):
    total = flat_idx.shape[0]
    per = total // NW
    mesh = plsc.VectorSubcoreMesh(core_axis_name="c", subcore_axis_name="s")
    cp = pltpu.CompilerParams()
    if "needs_layout_passes" in pltpu.CompilerParams.__dataclass_fields__:
        cp = dataclasses.replace(cp, needs_layout_passes=False)

    @functools.partial(
        pl.kernel,
        out_type=jax.ShapeDtypeStruct((total,), jnp.float32),
        mesh=mesh,
        compiler_params=cp,
        scratch_types=[
            pltpu.VMEM((TABLE,), jnp.float32),
            pltpu.VMEM((per,), jnp.int32),
            pltpu.VMEM((per,), jnp.float32),
        ],
    )
    def k(idx_hbm, ram_hbm, out_hbm, table_v, idx_v, out_v):
        wid = lax.axis_index("s") * 2 + lax.axis_index("c")
        base = wid * per
        pltpu.sync_copy(ram_hbm, table_v)
        pltpu.sync_copy(idx_hbm.at[pl.ds(base, per)], idx_v)

        @pl.loop(0, per, step=16)
        def _(i):
            iv = idx_v[pl.ds(i, 16)]
            out_v[pl.ds(i, 16)] = plsc.load_gather(table_v, [iv])

        pltpu.sync_copy(out_v, out_hbm.at[pl.ds(base, per)])

    return k(flat_idx, ram_flat)


def kernel(inp, connections, ram):
    b = inp.shape[0]
    flat_idx = _addresses(inp, connections)
    out = _lookup(flat_idx.reshape(-1), ram.reshape(-1))
    return out.reshape(b, N_NEURONS)


# speedup vs baseline: 162.7048x; 5.1245x over previous
import dataclasses
import functools

import jax
import jax.numpy as jnp
from jax import lax
from jax.experimental import pallas as pl
from jax.experimental.pallas import tpu as pltpu
from jax.experimental.pallas import tpu_sc as plsc

N_NEURONS = 6
N_BITS = 12
IN_BITS = 48
RAM_SIZE = 4096
TABLE = N_NEURONS * RAM_SIZE

CB = 8192
NW = 32


def _addr_body(conn_ref, inpt_ref, out_ref):
    conn = conn_ref[...]
    ii = lax.broadcasted_iota(jnp.int32, (N_NEURONS, N_BITS, IN_BITS), 2)
    kk = lax.broadcasted_iota(jnp.int32, (N_NEURONS, N_BITS, IN_BITS), 1)
    pow2 = jnp.left_shift(jnp.int32(1), (N_BITS - 1) - kk)
    w = jnp.sum(jnp.where(conn[:, :, None] == ii, pow2, 0), axis=1)
    w_hi = (w >> 6).astype(jnp.bfloat16)
    w_lo = (w & 63).astype(jnp.bfloat16)

    x = inpt_ref[...].astype(jnp.bfloat16)
    m_hi = jnp.dot(w_hi, x, preferred_element_type=jnp.float32)
    m_lo = jnp.dot(w_lo, x, preferred_element_type=jnp.float32)
    addr = (m_hi * 64.0 + m_lo).astype(jnp.int32)
    j = lax.broadcasted_iota(jnp.int32, (N_NEURONS, CB), 0)
    out_ref[...] = addr + j * RAM_SIZE


def _addresses(inp_t, connections):
    b = inp_t.shape[1]
    return pl.pallas_call(
        _addr_body,
        grid=(b // CB,),
        in_specs=[
            pl.BlockSpec((N_NEURONS, N_BITS), lambda i: (0, 0)),
            pl.BlockSpec((IN_BITS, CB), lambda i: (0, i)),
        ],
        out_specs=pl.BlockSpec((N_NEURONS, CB), lambda i: (0, i)),
        out_shape=jax.ShapeDtypeStruct((N_NEURONS, b), jnp.int32),
    )(connections, inp_t)


def _lookup(flat_idx, ram_flat):
    total = flat_idx.shape[0]
    per = total // NW
    mesh = plsc.VectorSubcoreMesh(core_axis_name="c", subcore_axis_name="s")
    cp = pltpu.CompilerParams()
    if "needs_layout_passes" in pltpu.CompilerParams.__dataclass_fields__:
        cp = dataclasses.replace(cp, needs_layout_passes=False)

    @functools.partial(
        pl.kernel,
        out_type=jax.ShapeDtypeStruct((total,), jnp.float32),
        mesh=mesh,
        compiler_params=cp,
        scratch_types=[
            pltpu.VMEM((TABLE,), jnp.float32),
            pltpu.VMEM((per,), jnp.int32),
            pltpu.VMEM((per,), jnp.float32),
        ],
    )
    def k(idx_hbm, ram_hbm, out_hbm, table_v, idx_v, out_v):
        wid = lax.axis_index("s") * 2 + lax.axis_index("c")
        base = wid * per
        pltpu.sync_copy(ram_hbm, table_v)
        pltpu.sync_copy(idx_hbm.at[pl.ds(base, per)], idx_v)

        @pl.loop(0, per, step=16)
        def _(i):
            iv = idx_v[pl.ds(i, 16)]
            out_v[pl.ds(i, 16)] = plsc.load_gather(table_v, [iv])

        pltpu.sync_copy(out_v, out_hbm.at[pl.ds(base, per)])

    return k(flat_idx, ram_flat)


def kernel(inp, connections, ram):
    b = inp.shape[0]
    flat_idx = _addresses(inp.T, connections).reshape(-1)
    out = _lookup(flat_idx, ram.reshape(-1))
    return out.reshape(N_NEURONS, b).T
